# Initial kernel scaffold; baseline (speedup 1.0000x reference)
import jax
import jax.numpy as jnp
from jax.experimental import pallas as pl


def kernel(src, pos_dst, neg_dst, t, raw_msg, memory, w_time, b_time, W_emb, b_emb, W1, b1, W2, b2):
    raise NotImplementedError("write your pallas kernel here")


# speedup vs baseline: 2.4594x; 2.4594x over previous
import functools

import jax
import jax.numpy as jnp
from jax import lax
from jax.experimental import pallas as pl
from jax.experimental.pallas import tpu as pltpu
from jax.experimental.pallas import tpu_sc as plsc

MEM_DIM = 128
EMB_DIM = 128


def _sc_gather(memory, idx_all):
    nrows = idx_all.shape[0]
    d = memory.shape[1]
    NW = 32
    per_w = nrows // NW
    CH = 128
    n_ch = per_w // CH
    mesh = plsc.VectorSubcoreMesh(core_axis_name="c", subcore_axis_name="s")

    @functools.partial(
        pl.kernel,
        mesh=mesh,
        out_type=jax.ShapeDtypeStruct((nrows, d), jnp.float32),
        scratch_types=[
            pltpu.VMEM((CH,), jnp.int32),
            pltpu.VMEM((CH, d), jnp.float32),
            pltpu.SemaphoreType.DMA,
        ],
    )
    def gather_k(table_hbm, idx_hbm, out_hbm, idx_v, rows_v, sem):
        wid = lax.axis_index("s") * 2 + lax.axis_index("c")
        base = wid * per_w

        def body(i, carry):
            off = base + i * CH
            pltpu.sync_copy(idx_hbm.at[pl.ds(off, CH)], idx_v)
            pltpu.async_copy(table_hbm.at[idx_v], rows_v, sem).wait()
            pltpu.sync_copy(rows_v, out_hbm.at[pl.ds(off, CH)])
            return carry

        lax.fori_loop(0, n_ch, body, 0)

    return gather_k(memory, idx_all)


def _tc_dense(gathered, t, w_time, b_time, W_emb, b_emb, W1, b1, W2, b2,
              interpret=False):
    b = t.shape[0]
    blk = 2048
    nblk = b // blk
    t2 = t.reshape(nblk, 1, blk)

    def body(t_ref, gs_ref, gp_ref, gn_ref, wt_ref, bt_ref, we_ref, be_ref,
             w1_ref, b1_ref, w2_ref, b2_ref, pos_ref, neg_ref):
        tb = t_ref[0, 0, :]
        tenc = jnp.cos(tb[:, None] * wt_ref[0, :][None, :]
                       + bt_ref[0, :][None, :])
        A = we_ref[:MEM_DIM, :]
        C = we_ref[MEM_DIM:, :]
        T = jnp.dot(tenc, C, preferred_element_type=jnp.float32) \
            + be_ref[0, :][None, :]
        es = jnp.dot(gs_ref[...], A, preferred_element_type=jnp.float32) + T
        ep = jnp.dot(gp_ref[...], A, preferred_element_type=jnp.float32) + T
        en = jnp.dot(gn_ref[...], A, preferred_element_type=jnp.float32) + T
        W1a = w1_ref[:EMB_DIM, :]
        W1b = w1_ref[EMB_DIM:, :]
        hs = jnp.dot(es, W1a, preferred_element_type=jnp.float32) \
            + b1_ref[0, :][None, :]
        hp = jnp.maximum(hs + jnp.dot(ep, W1b,
                                      preferred_element_type=jnp.float32), 0.0)
        hn = jnp.maximum(hs + jnp.dot(en, W1b,
                                      preferred_element_type=jnp.float32), 0.0)
        pos_ref[...] = jnp.dot(hp, w2_ref[...],
                               preferred_element_type=jnp.float32) + b2_ref[...]
        neg_ref[...] = jnp.dot(hn, w2_ref[...],
                               preferred_element_type=jnp.float32) + b2_ref[...]

    full = lambda shape: pl.BlockSpec(shape, lambda i: (0, 0))
    pos_out, neg_out = pl.pallas_call(
        body,
        grid=(nblk,),
        in_specs=[
            pl.BlockSpec((1, 1, blk), lambda i: (i, 0, 0)),
            pl.BlockSpec((blk, MEM_DIM), lambda i: (i, 0)),
            pl.BlockSpec((blk, MEM_DIM), lambda i: (i + nblk, 0)),
            pl.BlockSpec((blk, MEM_DIM), lambda i: (i + 2 * nblk, 0)),
            full(w_time.shape),
            full((1, b_time.shape[0])),
            full(W_emb.shape),
            full((1, b_emb.shape[0])),
            full(W1.shape),
            full((1, b1.shape[0])),
            full(W2.shape),
            full((1, 1)),
        ],
        out_specs=[
            pl.BlockSpec((blk, 1), lambda i: (i, 0)),
            pl.BlockSpec((blk, 1), lambda i: (i, 0)),
        ],
        out_shape=[
            jax.ShapeDtypeStruct((b, 1), jnp.float32),
            jax.ShapeDtypeStruct((b, 1), jnp.float32),
        ],
        interpret=interpret,
    )(t2, gathered, gathered, gathered, w_time, b_time.reshape(1, -1),
      W_emb, b_emb.reshape(1, -1), W1, b1.reshape(1, -1), W2,
      b2.reshape(1, 1))
    return pos_out, neg_out


def kernel(src, pos_dst, neg_dst, t, raw_msg, memory, w_time, b_time,
           W_emb, b_emb, W1, b1, W2, b2):
    idx_all = jnp.concatenate([src.astype(jnp.int32),
                               pos_dst.astype(jnp.int32),
                               neg_dst.astype(jnp.int32)])
    gathered = _sc_gather(memory, idx_all)
    return _tc_dense(gathered, t, w_time, b_time, W_emb, b_emb, W1, b1, W2, b2)


# speedup vs baseline: 2.9041x; 1.1808x over previous
import functools

import jax
import jax.numpy as jnp
from jax import lax
from jax.experimental import pallas as pl
from jax.experimental.pallas import tpu as pltpu
from jax.experimental.pallas import tpu_sc as plsc

MEM_DIM = 128
EMB_DIM = 128
NW = 32
CH = 128
NBUF = 4


def _sc_gather(memory, idx3):
    nw, n_ch, ch = idx3.shape
    d = memory.shape[1]
    nrows = nw * n_ch * ch
    mesh = plsc.VectorSubcoreMesh(core_axis_name="c", subcore_axis_name="s")

    @functools.partial(
        pl.kernel,
        mesh=mesh,
        out_type=jax.ShapeDtypeStruct((nrows, d), jnp.float32),
        scratch_types=[
            pltpu.VMEM((n_ch, ch), jnp.int32),
            *[pltpu.VMEM((ch, d), jnp.float32) for _ in range(NBUF)],
            *[pltpu.SemaphoreType.DMA for _ in range(2 * NBUF)],
        ],
    )
    def gather_k(table_hbm, idx_hbm, out_hbm, idx_v, *rest):
        bufs = rest[:NBUF]
        sems_g = rest[NBUF:2 * NBUF]
        sems_w = rest[2 * NBUF:]
        wid = lax.axis_index("s") * 2 + lax.axis_index("c")
        base = wid * (n_ch * ch)
        pltpu.sync_copy(idx_hbm.at[wid], idx_v)

        g_desc = [None] * NBUF
        w_desc = [None] * NBUF

        def start_gather(i):
            s = i % NBUF
            g_desc[s] = pltpu.async_copy(
                table_hbm.at[idx_v.at[i]], bufs[s], sems_g[s])

        def drain(i):
            s = i % NBUF
            g_desc[s].wait()
            w_desc[s] = pltpu.async_copy(
                bufs[s], out_hbm.at[pl.ds(base + i * ch, ch)], sems_w[s])

        look = NBUF - 1
        for i in range(n_ch):
            s = i % NBUF
            if i >= NBUF:
                w_desc[s].wait()
            start_gather(i)
            if i - look >= 0:
                drain(i - look)
        for j in range(max(0, n_ch - look), n_ch):
            drain(j)
        for j in range(max(0, n_ch - NBUF), n_ch):
            w_desc[j % NBUF].wait()

    return gather_k(memory, idx3)


def _tc_dense(gathered, t, w_time, b_time, W_emb, b_emb, W1, b1, W2, b2,
              interpret=False):
    b = t.shape[0]
    blk = 2048
    nblk = b // blk
    t2 = t.reshape(nblk, 1, blk)

    def body(t_ref, gs_ref, gp_ref, gn_ref, wt_ref, bt_ref, we_ref, be_ref,
             w1_ref, b1_ref, w2_ref, b2_ref, pos_ref, neg_ref,
             m1_ref, m2_ref, wtt_ref, bb_ref):
        @pl.when(pl.program_id(0) == 0)
        def _fold():
            A = we_ref[:MEM_DIM, :]
            C = we_ref[MEM_DIM:, :]
            W1a = w1_ref[:EMB_DIM, :]
            W1b = w1_ref[EMB_DIM:, :]
            W1s = W1a + W1b
            m1_ref[...] = jnp.dot(A, W1a, preferred_element_type=jnp.float32)
            m2_ref[...] = jnp.dot(A, W1b, preferred_element_type=jnp.float32)
            wtt_ref[...] = jnp.dot(C, W1s, preferred_element_type=jnp.float32)
            bb_ref[...] = jnp.dot(be_ref[...], W1s,
                                  preferred_element_type=jnp.float32) \
                + b1_ref[...]

        tb = t_ref[0, 0, :]
        tenc = jnp.cos(tb[:, None] * wt_ref[0, :][None, :]
                       + bt_ref[0, :][None, :])
        base = jnp.dot(tenc, wtt_ref[...],
                       preferred_element_type=jnp.float32) + bb_ref[...]
        gsm = jnp.dot(gs_ref[...], m1_ref[...],
                      preferred_element_type=jnp.float32) + base
        hp = jnp.maximum(gsm + jnp.dot(gp_ref[...], m2_ref[...],
                                       preferred_element_type=jnp.float32), 0.)
        hn = jnp.maximum(gsm + jnp.dot(gn_ref[...], m2_ref[...],
                                       preferred_element_type=jnp.float32), 0.)
        pos_ref[...] = jnp.dot(hp, w2_ref[...],
                               preferred_element_type=jnp.float32) + b2_ref[...]
        neg_ref[...] = jnp.dot(hn, w2_ref[...],
                               preferred_element_type=jnp.float32) + b2_ref[...]

    full = lambda shape: pl.BlockSpec(shape, lambda i: tuple(0 for _ in shape))
    pos_out, neg_out = pl.pallas_call(
        body,
        grid=(nblk,),
        in_specs=[
            pl.BlockSpec((1, 1, blk), lambda i: (i, 0, 0)),
            pl.BlockSpec((blk, MEM_DIM), lambda i: (i, 0)),
            pl.BlockSpec((blk, MEM_DIM), lambda i: (i + nblk, 0)),
            pl.BlockSpec((blk, MEM_DIM), lambda i: (i + 2 * nblk, 0)),
            full(w_time.shape),
            full((1, b_time.shape[0])),
            full(W_emb.shape),
            full((1, b_emb.shape[0])),
            full(W1.shape),
            full((1, b1.shape[0])),
            full(W2.shape),
            full((1, 1)),
        ],
        out_specs=[
            pl.BlockSpec((blk, 1), lambda i: (i, 0)),
            pl.BlockSpec((blk, 1), lambda i: (i, 0)),
        ],
        out_shape=[
            jax.ShapeDtypeStruct((b, 1), jnp.float32),
            jax.ShapeDtypeStruct((b, 1), jnp.float32),
        ],
        scratch_shapes=[
            pltpu.VMEM((MEM_DIM, EMB_DIM), jnp.float32),
            pltpu.VMEM((MEM_DIM, EMB_DIM), jnp.float32),
            pltpu.VMEM((w_time.shape[1], EMB_DIM), jnp.float32),
            pltpu.VMEM((1, EMB_DIM), jnp.float32),
        ],
        interpret=interpret,
    )(t2, gathered, gathered, gathered, w_time, b_time.reshape(1, -1),
      W_emb, b_emb.reshape(1, -1), W1, b1.reshape(1, -1), W2,
      b2.reshape(1, 1))
    return pos_out, neg_out


def kernel(src, pos_dst, neg_dst, t, raw_msg, memory, w_time, b_time,
           W_emb, b_emb, W1, b1, W2, b2):
    idx_all = jnp.concatenate([src.astype(jnp.int32),
                               pos_dst.astype(jnp.int32),
                               neg_dst.astype(jnp.int32)])
    n_ch = idx_all.shape[0] // (NW * CH)
    idx3 = idx_all.reshape(NW, n_ch, CH)
    gathered = _sc_gather(memory, idx3)
    return _tc_dense(gathered, t, w_time, b_time, W_emb, b_emb, W1, b1, W2, b2)


# speedup vs baseline: 3.6594x; 1.2601x over previous
import functools

import jax
import jax.numpy as jnp
from jax import lax
from jax.experimental import pallas as pl
from jax.experimental.pallas import tpu as pltpu
from jax.experimental.pallas import tpu_sc as plsc

MEM_DIM = 128
EMB_DIM = 128
NW = 32
CH = 128
NBUF = 4


def _sc_gather(memory, idx3):
    nw, n_ch, ch = idx3.shape
    d = memory.shape[1]
    nrows = nw * n_ch * ch
    mesh = plsc.VectorSubcoreMesh(core_axis_name="c", subcore_axis_name="s")

    @functools.partial(
        pl.kernel,
        mesh=mesh,
        out_type=jax.ShapeDtypeStruct((nrows, d), jnp.float32),
        scratch_types=[
            pltpu.VMEM((n_ch, ch), jnp.int32),
            *[pltpu.VMEM((ch, d), jnp.float32) for _ in range(NBUF)],
            *[pltpu.SemaphoreType.DMA for _ in range(2 * NBUF)],
        ],
    )
    def gather_k(table_hbm, idx_hbm, out_hbm, idx_v, *rest):
        bufs = rest[:NBUF]
        sems_g = rest[NBUF:2 * NBUF]
        sems_w = rest[2 * NBUF:]
        wid = lax.axis_index("s") * 2 + lax.axis_index("c")
        base = wid * (n_ch * ch)
        pltpu.sync_copy(idx_hbm.at[wid], idx_v)

        g_desc = [None] * NBUF
        w_desc = [None] * NBUF

        def start_gather(i):
            s = i % NBUF
            g_desc[s] = pltpu.async_copy(
                table_hbm.at[idx_v.at[i]], bufs[s], sems_g[s])

        def drain(i):
            s = i % NBUF
            g_desc[s].wait()
            w_desc[s] = pltpu.async_copy(
                bufs[s], out_hbm.at[pl.ds(base + i * ch, ch)], sems_w[s])

        look = NBUF - 1
        for i in range(n_ch):
            s = i % NBUF
            if i >= NBUF:
                w_desc[s].wait()
            start_gather(i)
            if i - look >= 0:
                drain(i - look)
        for j in range(max(0, n_ch - look), n_ch):
            drain(j)
        for j in range(max(0, n_ch - NBUF), n_ch):
            w_desc[j % NBUF].wait()

    return gather_k(memory, idx3)


def _tc_dense(gathered, t, w_time, b_time, W_emb, b_emb, W1, b1, W2, b2,
              interpret=False):
    b = t.shape[0]
    blk = 2048
    nblk = b // blk
    t2 = t.reshape(nblk, 1, blk)

    def body(t_ref, gs_ref, gp_ref, gn_ref, wt_ref, bt_ref, we_ref, be_ref,
             w1_ref, b1_ref, w2_ref, b2_ref, pos_ref, neg_ref,
             m1_ref, m2_ref, wtt_ref, bb_ref):
        @pl.when(pl.program_id(0) == 0)
        def _fold():
            A = we_ref[:MEM_DIM, :]
            C = we_ref[MEM_DIM:, :]
            W1a = w1_ref[:EMB_DIM, :]
            W1b = w1_ref[EMB_DIM:, :]
            W1s = W1a + W1b
            m1_ref[...] = jnp.dot(A, W1a, preferred_element_type=jnp.float32)
            m2_ref[...] = jnp.dot(A, W1b, preferred_element_type=jnp.float32)
            wtt_ref[...] = jnp.dot(C, W1s, preferred_element_type=jnp.float32)
            bb_ref[...] = jnp.dot(be_ref[...], W1s,
                                  preferred_element_type=jnp.float32) \
                + b1_ref[...]

        tb = t_ref[0, 0, :]
        tenc_t = jnp.cos(wt_ref[0, :][:, None] * tb[None, :]
                         + bt_ref[0, :][:, None])
        base = jax.lax.dot_general(
            tenc_t, wtt_ref[...],
            dimension_numbers=(((0,), (0,)), ((), ())),
            preferred_element_type=jnp.float32) + bb_ref[...]
        gsm = jnp.dot(gs_ref[...], m1_ref[...],
                      preferred_element_type=jnp.float32) + base
        hp = jnp.maximum(gsm + jnp.dot(gp_ref[...], m2_ref[...],
                                       preferred_element_type=jnp.float32), 0.)
        hn = jnp.maximum(gsm + jnp.dot(gn_ref[...], m2_ref[...],
                                       preferred_element_type=jnp.float32), 0.)
        pos_ref[...] = jnp.dot(hp, w2_ref[...],
                               preferred_element_type=jnp.float32) + b2_ref[...]
        neg_ref[...] = jnp.dot(hn, w2_ref[...],
                               preferred_element_type=jnp.float32) + b2_ref[...]

    full = lambda shape: pl.BlockSpec(shape, lambda i: tuple(0 for _ in shape))
    pos_out, neg_out = pl.pallas_call(
        body,
        grid=(nblk,),
        in_specs=[
            pl.BlockSpec((1, 1, blk), lambda i: (i, 0, 0)),
            pl.BlockSpec((blk, MEM_DIM), lambda i: (i, 0)),
            pl.BlockSpec((blk, MEM_DIM), lambda i: (i + nblk, 0)),
            pl.BlockSpec((blk, MEM_DIM), lambda i: (i + 2 * nblk, 0)),
            full(w_time.shape),
            full((1, b_time.shape[0])),
            full(W_emb.shape),
            full((1, b_emb.shape[0])),
            full(W1.shape),
            full((1, b1.shape[0])),
            full(W2.shape),
            full((1, 1)),
        ],
        out_specs=[
            pl.BlockSpec((blk, 1), lambda i: (i, 0)),
            pl.BlockSpec((blk, 1), lambda i: (i, 0)),
        ],
        out_shape=[
            jax.ShapeDtypeStruct((b, 1), jnp.float32),
            jax.ShapeDtypeStruct((b, 1), jnp.float32),
        ],
        scratch_shapes=[
            pltpu.VMEM((MEM_DIM, EMB_DIM), jnp.float32),
            pltpu.VMEM((MEM_DIM, EMB_DIM), jnp.float32),
            pltpu.VMEM((w_time.shape[1], EMB_DIM), jnp.float32),
            pltpu.VMEM((1, EMB_DIM), jnp.float32),
        ],
        interpret=interpret,
    )(t2, gathered, gathered, gathered, w_time, b_time.reshape(1, -1),
      W_emb, b_emb.reshape(1, -1), W1, b1.reshape(1, -1), W2,
      b2.reshape(1, 1))
    return pos_out, neg_out


def kernel(src, pos_dst, neg_dst, t, raw_msg, memory, w_time, b_time,
           W_emb, b_emb, W1, b1, W2, b2):
    idx_all = jnp.concatenate([src.astype(jnp.int32),
                               pos_dst.astype(jnp.int32),
                               neg_dst.astype(jnp.int32)])
    n_ch = idx_all.shape[0] // (NW * CH)
    idx3 = idx_all.reshape(NW, n_ch, CH)
    gathered = _sc_gather(memory, idx3)
    return _tc_dense(gathered, t, w_time, b_time, W_emb, b_emb, W1, b1, W2, b2)


# speedup vs baseline: 4.3320x; 1.1838x over previous
import functools

import jax
import jax.numpy as jnp
from jax import lax
from jax.experimental import pallas as pl
from jax.experimental.pallas import tpu as pltpu
from jax.experimental.pallas import tpu_sc as plsc

MEM_DIM = 128
EMB_DIM = 128
NW = 32
CH = 128
NBUF = 4


def _sc_gather(memory, src, pos_dst, neg_dst):
    b = src.shape[0]
    d = memory.shape[1]
    nrows = 3 * b
    per_sec = b // NW
    sec_ch = per_sec // CH
    n_ch = 3 * sec_ch
    mesh = plsc.VectorSubcoreMesh(core_axis_name="c", subcore_axis_name="s")

    @functools.partial(
        pl.kernel,
        mesh=mesh,
        out_type=jax.ShapeDtypeStruct((nrows, d), jnp.float32),
        scratch_types=[
            pltpu.VMEM((n_ch, CH), jnp.int32),
            *[pltpu.VMEM((CH, d), jnp.float32) for _ in range(NBUF)],
            *[pltpu.SemaphoreType.DMA for _ in range(2 * NBUF)],
        ],
    )
    def gather_k(table_hbm, src_hbm, pos_hbm, neg_hbm, out_hbm, idx_v, *rest):
        bufs = rest[:NBUF]
        sems_g = rest[NBUF:2 * NBUF]
        sems_w = rest[2 * NBUF:]
        wid = lax.axis_index("s") * 2 + lax.axis_index("c")

        idx_srcs = (src_hbm, pos_hbm, neg_hbm)
        def out_off(i):
            sec, c = divmod(i, sec_ch)
            return sec * b + wid * per_sec + c * CH

        for sec in range(3):
            for c in range(sec_ch):
                pltpu.sync_copy(
                    idx_srcs[sec].at[pl.ds(wid * per_sec + c * CH, CH)],
                    idx_v.at[sec * sec_ch + c])

        g_desc = [None] * NBUF
        w_desc = [None] * NBUF

        def start_gather(i):
            s = i % NBUF
            g_desc[s] = pltpu.async_copy(
                table_hbm.at[idx_v.at[i]], bufs[s], sems_g[s])

        def drain(i):
            s = i % NBUF
            g_desc[s].wait()
            w_desc[s] = pltpu.async_copy(
                bufs[s], out_hbm.at[pl.ds(out_off(i), CH)], sems_w[s])

        look = NBUF - 1
        for i in range(n_ch):
            s = i % NBUF
            if i >= NBUF:
                w_desc[s].wait()
            start_gather(i)
            if i - look >= 0:
                drain(i - look)
        for j in range(max(0, n_ch - look), n_ch):
            drain(j)
        for j in range(max(0, n_ch - NBUF), n_ch):
            w_desc[j % NBUF].wait()

    return gather_k(memory, src, pos_dst, neg_dst)


def _tc_dense(gathered, t, w_time, b_time, W_emb, b_emb, W1, b1, W2, b2,
              interpret=False):
    b = t.shape[0]
    blk = 2048
    nblk = b // blk
    t2 = t.reshape(nblk, 1, blk)

    def body(t_ref, gs_ref, gp_ref, gn_ref, wt_ref, bt_ref, we_ref, be_ref,
             w1_ref, b1_ref, w2_ref, b2_ref, pos_ref, neg_ref,
             m1_ref, m2_ref, wtt_ref, bb_ref):
        @pl.when(pl.program_id(0) == 0)
        def _fold():
            A = we_ref[:MEM_DIM, :]
            C = we_ref[MEM_DIM:, :]
            W1a = w1_ref[:EMB_DIM, :]
            W1b = w1_ref[EMB_DIM:, :]
            W1s = W1a + W1b
            m1_ref[...] = jnp.dot(A, W1a, preferred_element_type=jnp.float32)
            m2_ref[...] = jnp.dot(A, W1b, preferred_element_type=jnp.float32)
            wtt_ref[...] = jnp.dot(C, W1s, preferred_element_type=jnp.float32)
            bb_ref[...] = jnp.dot(be_ref[...], W1s,
                                  preferred_element_type=jnp.float32) \
                + b1_ref[...]

        tb = t_ref[0, 0, :]
        tenc_t = jnp.cos(wt_ref[0, :][:, None] * tb[None, :]
                         + bt_ref[0, :][:, None])
        base = jax.lax.dot_general(
            tenc_t, wtt_ref[...],
            dimension_numbers=(((0,), (0,)), ((), ())),
            preferred_element_type=jnp.float32) + bb_ref[...]
        gsm = jnp.dot(gs_ref[...], m1_ref[...],
                      preferred_element_type=jnp.float32) + base
        hp = jnp.maximum(gsm + jnp.dot(gp_ref[...], m2_ref[...],
                                       preferred_element_type=jnp.float32), 0.)
        hn = jnp.maximum(gsm + jnp.dot(gn_ref[...], m2_ref[...],
                                       preferred_element_type=jnp.float32), 0.)
        pos_ref[...] = jax.lax.dot_general(
            w2_ref[...], hp, dimension_numbers=(((0,), (1,)), ((), ())),
            preferred_element_type=jnp.float32) + b2_ref[...]
        neg_ref[...] = jax.lax.dot_general(
            w2_ref[...], hn, dimension_numbers=(((0,), (1,)), ((), ())),
            preferred_element_type=jnp.float32) + b2_ref[...]

    full = lambda shape: pl.BlockSpec(shape, lambda i: tuple(0 for _ in shape))
    pos_t, neg_t = pl.pallas_call(
        body,
        grid=(nblk,),
        in_specs=[
            pl.BlockSpec((1, 1, blk), lambda i: (i, 0, 0)),
            pl.BlockSpec((blk, MEM_DIM), lambda i: (i, 0)),
            pl.BlockSpec((blk, MEM_DIM), lambda i: (i + nblk, 0)),
            pl.BlockSpec((blk, MEM_DIM), lambda i: (i + 2 * nblk, 0)),
            full(w_time.shape),
            full((1, b_time.shape[0])),
            full(W_emb.shape),
            full((1, b_emb.shape[0])),
            full(W1.shape),
            full((1, b1.shape[0])),
            full(W2.shape),
            full((1, 1)),
        ],
        out_specs=[
            pl.BlockSpec((1, blk), lambda i: (0, i)),
            pl.BlockSpec((1, blk), lambda i: (0, i)),
        ],
        out_shape=[
            jax.ShapeDtypeStruct((1, b), jnp.float32),
            jax.ShapeDtypeStruct((1, b), jnp.float32),
        ],
        scratch_shapes=[
            pltpu.VMEM((MEM_DIM, EMB_DIM), jnp.float32),
            pltpu.VMEM((MEM_DIM, EMB_DIM), jnp.float32),
            pltpu.VMEM((w_time.shape[1], EMB_DIM), jnp.float32),
            pltpu.VMEM((1, EMB_DIM), jnp.float32),
        ],
        interpret=interpret,
    )(t2, gathered, gathered, gathered, w_time, b_time.reshape(1, -1),
      W_emb, b_emb.reshape(1, -1), W1, b1.reshape(1, -1), W2,
      b2.reshape(1, 1))
    return pos_t.reshape(b, 1), neg_t.reshape(b, 1)


def kernel(src, pos_dst, neg_dst, t, raw_msg, memory, w_time, b_time,
           W_emb, b_emb, W1, b1, W2, b2):
    gathered = _sc_gather(memory, src.astype(jnp.int32),
                          pos_dst.astype(jnp.int32),
                          neg_dst.astype(jnp.int32))
    return _tc_dense(gathered, t, w_time, b_time, W_emb, b_emb, W1, b1, W2, b2)
